# Initial kernel scaffold; baseline (speedup 1.0000x reference)
import jax
import jax.numpy as jnp
from jax.experimental import pallas as pl


def kernel(input, signals_weight, channels_weight):
    raise NotImplementedError("write your pallas kernel here")


# speedup vs baseline: 10.1960x; 10.1960x over previous
import jax
import jax.numpy as jnp
from jax.experimental import pallas as pl

MAXV = 52000.0
MINV = -53000.0
LEV = 1000
NGRAM = 4
D = 4096
C = 23
T = 64
B = 16
ROWS_BLK = 200


def _thresh_body(s_ref, base_ref, t_ref):
    i = pl.program_id(0)

    @pl.when(i == 0)
    def _():
        t_ref[...] = jnp.zeros_like(t_ref)

    s = s_ref[...]
    base = base_ref[...]
    t_ref[...] += jnp.sum((s * base > 0.0).astype(jnp.float32), axis=0,
                          keepdims=True)


def _encode_body(idx_ref, ch_ref, t_ref, base_ref, out_ref):
    idx = idx_ref[0]
    t = t_ref[...]
    q = jnp.zeros((T, D), dtype=jnp.float32)
    for c in range(C):
        col = idx[:, c][:, None]
        row = ch_ref[c][None, :]
        q = q + jnp.where(col >= t, -row, row)
    samples = q * base_ref[...]

    def roll1(a):
        return jnp.concatenate([a[:, -1:], a[:, :-1]], axis=1)

    r0 = samples
    r1 = roll1(r0)
    r2 = roll1(r1)
    r3 = roll1(r2)
    w = T - (NGRAM - 1)
    ng = (r3[0:w] * r2[1:w + 1] * r1[2:w + 2] * r0[3:w + 3])
    s = jnp.sum(ng, axis=0, keepdims=True)
    out_ref[...] = jnp.where(s > 0.0, 1.0, -1.0).astype(jnp.float32)[None]


def kernel(input, signals_weight, channels_weight):
    idxf = jnp.round((input - MINV) / (MAXV - MINV) * (LEV - 1))
    idxf = jnp.clip(idxf, 0.0, float(LEV - 1)).astype(jnp.float32)

    t = pl.pallas_call(
        _thresh_body,
        grid=(LEV // ROWS_BLK,),
        in_specs=[
            pl.BlockSpec((ROWS_BLK, D), lambda i: (i, 0)),
            pl.BlockSpec((1, D), lambda i: (0, 0)),
        ],
        out_specs=pl.BlockSpec((1, D), lambda i: (0, 0)),
        out_shape=jax.ShapeDtypeStruct((1, D), jnp.float32),
    )(signals_weight, signals_weight[0:1])

    out = pl.pallas_call(
        _encode_body,
        grid=(B,),
        in_specs=[
            pl.BlockSpec((1, T, C), lambda i: (i, 0, 0)),
            pl.BlockSpec((C, D), lambda i: (0, 0)),
            pl.BlockSpec((1, D), lambda i: (0, 0)),
            pl.BlockSpec((1, D), lambda i: (0, 0)),
        ],
        out_specs=pl.BlockSpec((1, 1, D), lambda i: (i, 0, 0)),
        out_shape=jax.ShapeDtypeStruct((B, 1, D), jnp.float32),
    )(idxf, channels_weight, t, signals_weight[0:1])
    return out.reshape(B, D)
